# baseline (speedup 1.0000x reference)
import jax
import jax.numpy as jnp
from jax import lax
from jax.experimental import pallas as pl
from jax.experimental.pallas import tpu as pltpu
from jax.experimental.pallas import tpu_sc as plsc

BATCH = 32
N_ATOMS = 1024
ATOM = 512
SLOTS = 64
N_SAMPLES = 32768
CHUNK = 64
NCHUNK = N_ATOMS // CHUNK
NC = 2
NS = 16
LANES = 16
GROUPS = ATOM // 32


CHUNK_ELEMS = CHUNK * ATOM
CHUNK_WORDS = CHUNK * ATOM // 2


def _body(x_hbm, atoms_hbm, out_hbm, idx_v, buf0, buf1, acc_v, idx_s, sems):
    sid = lax.axis_index("s")
    b = sid * NC + lax.axis_index("c")

    pltpu.sync_copy(x_hbm.at[b], idx_v)

    pltpu.async_copy(atoms_hbm.at[pl.ds(0, CHUNK)], buf0, sems.at[0])

    def stage_body(g, carry):
        idxvec = idx_v[pl.ds(g * LANES, LANES)]
        for k in range(LANES):
            idx_s[g * LANES + k] = idxvec[k]
        return carry

    lax.fori_loop(0, N_ATOMS // LANES, stage_body, 0)

    z = jnp.zeros((LANES,), jnp.float32)

    @plsc.parallel_loop(0, SLOTS)
    def _zero(i):
        base = i * ATOM
        for j in range(ATOM // LANES):
            acc_v[pl.ds(base + j * LANES, LANES)] = z

    def accum_chunk(c, buf):
        @plsc.parallel_loop(0, CHUNK, unroll=8)
        def _accum(a):
            base = idx_s[c * CHUNK + a] * ATOM
            for t in range(GROUPS):
                w = buf[a, pl.ds(t * LANES, LANES)]
                lo = lax.bitcast_convert_type(w << 16, jnp.float32)
                hi = lax.bitcast_convert_type(w & jnp.int32(-65536), jnp.float32)
                plsc.addupdate(acc_v.at[pl.ds(base + t * 32, LANES)], lo)
                plsc.addupdate(
                    acc_v.at[pl.ds(base + t * 32 + LANES, LANES)], hi
                )

    def pair_body(i, carry):
        c0 = 2 * i
        pltpu.async_copy(
            atoms_hbm.at[pl.ds((c0 + 1) * CHUNK, CHUNK)],
            buf1,
            sems.at[1],
        )
        pltpu.make_async_copy(
            atoms_hbm.at[pl.ds(c0 * CHUNK, CHUNK)], buf0, sems.at[0]
        ).wait()
        accum_chunk(c0, buf0)

        @pl.when(c0 + 2 < NCHUNK)
        def _start_next():
            pltpu.async_copy(
                atoms_hbm.at[pl.ds((c0 + 2) * CHUNK, CHUNK)],
                buf0,
                sems.at[0],
            )

        pltpu.make_async_copy(
            atoms_hbm.at[pl.ds((c0 + 1) * CHUNK, CHUNK)],
            buf1,
            sems.at[1],
        ).wait()
        accum_chunk(c0 + 1, buf1)
        return carry

    lax.fori_loop(0, NCHUNK // 2, pair_body, 0)

    pltpu.sync_copy(acc_v, out_hbm.at[b, 0])


def kernel(x, atoms):
    ab = atoms.reshape(N_ATOMS, GROUPS, 2, LANES).astype(jnp.bfloat16)
    u = lax.bitcast_convert_type(ab, jnp.uint16).astype(jnp.int32)
    ar = (u[:, :, 0, :] | (u[:, :, 1, :] << 16)).reshape(N_ATOMS, ATOM // 2)
    mesh = plsc.VectorSubcoreMesh(core_axis_name="c", subcore_axis_name="s")
    f = pl.kernel(
        _body,
        out_type=jax.ShapeDtypeStruct((BATCH, 1, N_SAMPLES), jnp.float32),
        mesh=mesh,
        scratch_types=[
            pltpu.VMEM((N_ATOMS,), jnp.int32),
            pltpu.VMEM((CHUNK, ATOM // 2), jnp.int32),
            pltpu.VMEM((CHUNK, ATOM // 2), jnp.int32),
            pltpu.VMEM((SLOTS * ATOM,), jnp.float32),
            pltpu.SMEM((N_ATOMS,), jnp.int32),
            pltpu.SemaphoreType.DMA((2,)),
        ],
    )
    return f(x, ar)


# speedup vs baseline: 1.1181x; 1.1181x over previous
import jax
import jax.numpy as jnp
from jax import lax
from jax.experimental import pallas as pl
from jax.experimental.pallas import tpu as pltpu
from jax.experimental.pallas import tpu_sc as plsc

BATCH = 32
N_ATOMS = 1024
ATOM = 512
SLOTS = 64
N_SAMPLES = 32768
CHUNK = 64
NCHUNK = N_ATOMS // CHUNK
NC = 2
NS = 16
LANES = 16
GROUPS = ATOM // 32


CHUNK_ELEMS = CHUNK * ATOM
CHUNK_WORDS = CHUNK * ATOM // 2


def _body(x_hbm, atoms_hbm, out_hbm, idx_v, buf0, buf1, acc_v, idx_s, sems):
    sid = lax.axis_index("s")
    b = sid * NC + lax.axis_index("c")

    pltpu.sync_copy(x_hbm.at[b], idx_v)

    pltpu.async_copy(atoms_hbm.at[pl.ds(0, CHUNK)], buf0, sems.at[0])

    def stage_body(g, carry):
        idxvec = idx_v[pl.ds(g * LANES, LANES)]
        for k in range(LANES):
            idx_s[g * LANES + k] = idxvec[k]
        return carry

    lax.fori_loop(0, N_ATOMS // LANES, stage_body, 0)

    z = jnp.zeros((LANES,), jnp.float32)

    @plsc.parallel_loop(0, SLOTS)
    def _zero(i):
        base = i * ATOM
        for j in range(ATOM // LANES):
            acc_v[pl.ds(base + j * LANES, LANES)] = z

    def accum_chunk(c, buf):
        @plsc.parallel_loop(0, CHUNK, unroll=4)
        def _accum(a):
            base = idx_s[c * CHUNK + a] * ATOM
            for t in range(GROUPS):
                w = buf[a, pl.ds(t * LANES, LANES)]
                lo = lax.bitcast_convert_type(w << 16, jnp.float32)
                hi = lax.bitcast_convert_type(w & jnp.int32(-65536), jnp.float32)
                plsc.addupdate(acc_v.at[pl.ds(base + t * 32, LANES)], lo)
                plsc.addupdate(
                    acc_v.at[pl.ds(base + t * 32 + LANES, LANES)], hi
                )

    def pair_body(i, carry):
        c0 = 2 * i
        pltpu.async_copy(
            atoms_hbm.at[pl.ds((c0 + 1) * CHUNK, CHUNK)],
            buf1,
            sems.at[1],
        )
        pltpu.make_async_copy(
            atoms_hbm.at[pl.ds(c0 * CHUNK, CHUNK)], buf0, sems.at[0]
        ).wait()
        accum_chunk(c0, buf0)

        @pl.when(c0 + 2 < NCHUNK)
        def _start_next():
            pltpu.async_copy(
                atoms_hbm.at[pl.ds((c0 + 2) * CHUNK, CHUNK)],
                buf0,
                sems.at[0],
            )

        pltpu.make_async_copy(
            atoms_hbm.at[pl.ds((c0 + 1) * CHUNK, CHUNK)],
            buf1,
            sems.at[1],
        ).wait()
        accum_chunk(c0 + 1, buf1)
        return carry

    lax.fori_loop(0, NCHUNK // 2, pair_body, 0)

    pltpu.sync_copy(acc_v, out_hbm.at[b, 0])


def kernel(x, atoms):
    ab = atoms.reshape(N_ATOMS, GROUPS, 2, LANES).astype(jnp.bfloat16)
    u = lax.bitcast_convert_type(ab, jnp.uint16).astype(jnp.int32)
    ar = (u[:, :, 0, :] | (u[:, :, 1, :] << 16)).reshape(N_ATOMS, ATOM // 2)
    mesh = plsc.VectorSubcoreMesh(core_axis_name="c", subcore_axis_name="s")
    f = pl.kernel(
        _body,
        out_type=jax.ShapeDtypeStruct((BATCH, 1, N_SAMPLES), jnp.float32),
        mesh=mesh,
        scratch_types=[
            pltpu.VMEM((N_ATOMS,), jnp.int32),
            pltpu.VMEM((CHUNK, ATOM // 2), jnp.int32),
            pltpu.VMEM((CHUNK, ATOM // 2), jnp.int32),
            pltpu.VMEM((SLOTS * ATOM,), jnp.float32),
            pltpu.SMEM((N_ATOMS,), jnp.int32),
            pltpu.SemaphoreType.DMA((2,)),
        ],
    )
    return f(x, ar)
